# baseline (speedup 1.0000x reference)
import functools

import jax
import jax.numpy as jnp
from jax import lax
from jax.experimental import pallas as pl
from jax.experimental.pallas import tpu as pltpu
from jax.experimental.pallas import tpu_sc as plsc

B, N, D = 4, 8192, 1024
BN = B * N

NC, NS = 2, 16
NW = NC * NS
ROWS_PER_W = BN // NW
CHUNK = 32
NCHUNK = ROWS_PER_W // CHUNK


_KA_ROWS = 512


def _keys_body(x_ref, k_ref):
    xb = x_ref[0]
    acc = xb[:, 0:128]
    for j in range(1, 8):
        acc = acc + xb[:, 128 * j:128 * (j + 1)]
    b = acc[:, 0:8]
    for m in range(1, 16):
        b = b + acc[:, 8 * m:8 * (m + 1)]
    b = b[:, 0:4] + b[:, 4:8]
    b = b[:, 0:2] + b[:, 2:4]
    tot = b[:, 0] + b[:, 1]
    k_ref[0, 0, :] = xb[:, 0] + tot


def _compute_keys(x):
    nblk = N // _KA_ROWS
    out = pl.pallas_call(
        _keys_body,
        grid=(B, nblk),
        in_specs=[pl.BlockSpec((1, _KA_ROWS, D), lambda b, i: (b, i, 0))],
        out_specs=pl.BlockSpec((1, 1, _KA_ROWS), lambda b, i: (b * nblk + i, 0, 0)),
        out_shape=jax.ShapeDtypeStruct((B * nblk, 1, _KA_ROWS), jnp.float32),
    )(x)
    return out.reshape(B, N)


_RB = 1024


def _rank_body(ki_ref, kj_ref, out_ref, acc_ref):
    bidx = pl.program_id(0)
    i = pl.program_id(1)
    j = pl.program_id(2)
    ki = ki_ref[0]
    kj = kj_ref[0]
    lt = (kj < ki).astype(jnp.float32)
    eq = kj == ki
    gi = i * _RB + lax.broadcasted_iota(jnp.int32, (_RB, _RB), 1)
    gj = j * _RB + lax.broadcasted_iota(jnp.int32, (_RB, _RB), 0)
    tie = jnp.where(eq & (gj < gi), 1.0, 0.0)
    s = jnp.sum(lt + tie, axis=0, keepdims=True)

    @pl.when(j == 0)
    def _():
        acc_ref[...] = s

    @pl.when(j > 0)
    def _():
        acc_ref[...] = acc_ref[...] + s

    @pl.when(j == pl.num_programs(2) - 1)
    def _():
        out_ref[0] = acc_ref[...].astype(jnp.int32) + bidx * N


def _compute_ranks(keys):
    keys_col = keys.reshape(B, N, 1)
    nb = N // _RB
    keys_row = keys.reshape(B * nb, 1, _RB)
    out = pl.pallas_call(
        _rank_body,
        grid=(B, nb, nb),
        in_specs=[
            pl.BlockSpec((1, 1, _RB), lambda b, i, j: (b * nb + i, 0, 0)),
            pl.BlockSpec((1, _RB, 1), lambda b, i, j: (b, j, 0)),
        ],
        out_specs=pl.BlockSpec((1, 1, _RB), lambda b, i, j: (b * nb + i, 0, 0)),
        out_shape=jax.ShapeDtypeStruct((B * nb, 1, _RB), jnp.int32),
        scratch_shapes=[pltpu.VMEM((1, _RB), jnp.float32)],
    )(keys_row, keys_col)
    return out.reshape(B, N)


@functools.cache
def _build_sc_scatter():
    mesh = plsc.VectorSubcoreMesh(core_axis_name="c", subcore_axis_name="s")

    @functools.partial(
        pl.kernel,
        mesh=mesh,
        out_type=jax.ShapeDtypeStruct((BN, D), jnp.float32),
        scratch_types=[
            pltpu.VMEM((CHUNK, D), jnp.float32),
            pltpu.VMEM((CHUNK,), jnp.int32),
            pltpu.SemaphoreType.DMA,
        ],
    )
    def _sc_scatter(x_hbm, gidx_hbm, out_hbm, rows_v, idx_v, sem):
        wid = lax.axis_index("s") * NC + lax.axis_index("c")
        base = wid * ROWS_PER_W

        def body(g, carry):
            start = base + g * CHUNK
            pltpu.sync_copy(gidx_hbm.at[pl.ds(start, CHUNK)], idx_v)
            pltpu.sync_copy(x_hbm.at[pl.ds(start, CHUNK)], rows_v)
            pltpu.async_copy(rows_v, out_hbm.at[idx_v], sem).wait()
            return carry

        lax.fori_loop(0, NCHUNK, body, 0)

    return _sc_scatter


def kernel(x):
    keys = _compute_keys(x)
    gidx = _compute_ranks(keys)
    out2d = _build_sc_scatter()(x.reshape(BN, D), gidx.reshape(BN))
    return out2d.reshape(B, N, D)


# speedup vs baseline: 1.1408x; 1.1408x over previous
import functools

import jax
import jax.numpy as jnp
from jax import lax
from jax.experimental import pallas as pl
from jax.experimental.pallas import tpu as pltpu
from jax.experimental.pallas import tpu_sc as plsc

B, N, D = 4, 8192, 1024
BN = B * N

NC, NS = 2, 16
NW = NC * NS
ROWS_PER_W = BN // NW
CHUNK = 32
NCHUNK = ROWS_PER_W // CHUNK


_KA_ROWS = 512


def _keys_body(x_ref, k_ref):
    xb = x_ref[0]
    acc = xb[:, 0:128]
    for j in range(1, 8):
        acc = acc + xb[:, 128 * j:128 * (j + 1)]
    b = acc[:, 0:8]
    for m in range(1, 16):
        b = b + acc[:, 8 * m:8 * (m + 1)]
    b = b[:, 0:4] + b[:, 4:8]
    b = b[:, 0:2] + b[:, 2:4]
    tot = b[:, 0] + b[:, 1]
    k_ref[0, 0, :] = xb[:, 0] + tot


def _compute_keys(x):
    nblk = N // _KA_ROWS
    out = pl.pallas_call(
        _keys_body,
        grid=(B, nblk),
        in_specs=[pl.BlockSpec((1, _KA_ROWS, D), lambda b, i: (b, i, 0))],
        out_specs=pl.BlockSpec((1, 1, _KA_ROWS), lambda b, i: (b * nblk + i, 0, 0)),
        out_shape=jax.ShapeDtypeStruct((B * nblk, 1, _KA_ROWS), jnp.float32),
    )(x)
    return out.reshape(B, N)


_RB = 1024


def _rank_body(ki_ref, kj_ref, out_ref, acc_ref):
    bidx = pl.program_id(0)
    i = pl.program_id(1)
    j = pl.program_id(2)
    ki = ki_ref[0]
    kj = kj_ref[0]

    @pl.when(j == 0)
    def _():
        acc_ref[...] = jnp.zeros((1, _RB), jnp.float32)

    @pl.when(j < i)
    def _():
        c = (kj <= ki).astype(jnp.float32)
        acc_ref[...] = acc_ref[...] + jnp.sum(c, axis=0, keepdims=True)

    @pl.when(j > i)
    def _():
        c = (kj < ki).astype(jnp.float32)
        acc_ref[...] = acc_ref[...] + jnp.sum(c, axis=0, keepdims=True)

    @pl.when(j == i)
    def _():
        lt = (kj < ki).astype(jnp.float32)
        eq = kj == ki
        qi = lax.broadcasted_iota(jnp.int32, (_RB, _RB), 1)
        qj = lax.broadcasted_iota(jnp.int32, (_RB, _RB), 0)
        tie = jnp.where(eq & (qj < qi), 1.0, 0.0)
        acc_ref[...] = acc_ref[...] + jnp.sum(lt + tie, axis=0, keepdims=True)

    @pl.when(j == pl.num_programs(2) - 1)
    def _():
        out_ref[0] = acc_ref[...].astype(jnp.int32) + bidx * N


def _compute_ranks(keys):
    keys_col = keys.reshape(B, N, 1)
    nb = N // _RB
    keys_row = keys.reshape(B * nb, 1, _RB)
    out = pl.pallas_call(
        _rank_body,
        grid=(B, nb, nb),
        in_specs=[
            pl.BlockSpec((1, 1, _RB), lambda b, i, j: (b * nb + i, 0, 0)),
            pl.BlockSpec((1, _RB, 1), lambda b, i, j: (b, j, 0)),
        ],
        out_specs=pl.BlockSpec((1, 1, _RB), lambda b, i, j: (b * nb + i, 0, 0)),
        out_shape=jax.ShapeDtypeStruct((B * nb, 1, _RB), jnp.int32),
        scratch_shapes=[pltpu.VMEM((1, _RB), jnp.float32)],
    )(keys_row, keys_col)
    return out.reshape(B, N)


@functools.cache
def _build_sc_scatter():
    mesh = plsc.VectorSubcoreMesh(core_axis_name="c", subcore_axis_name="s")

    @functools.partial(
        pl.kernel,
        mesh=mesh,
        out_type=jax.ShapeDtypeStruct((BN, D), jnp.float32),
        scratch_types=[
            pltpu.VMEM((CHUNK, D), jnp.float32),
            pltpu.VMEM((CHUNK,), jnp.int32),
            pltpu.SemaphoreType.DMA,
        ],
    )
    def _sc_scatter(x_hbm, gidx_hbm, out_hbm, rows_v, idx_v, sem):
        wid = lax.axis_index("s") * NC + lax.axis_index("c")
        base = wid * ROWS_PER_W

        def body(g, carry):
            start = base + g * CHUNK
            pltpu.sync_copy(gidx_hbm.at[pl.ds(start, CHUNK)], idx_v)
            pltpu.sync_copy(x_hbm.at[pl.ds(start, CHUNK)], rows_v)
            pltpu.async_copy(rows_v, out_hbm.at[idx_v], sem).wait()
            return carry

        lax.fori_loop(0, NCHUNK, body, 0)

    return _sc_scatter


def kernel(x):
    keys = _compute_keys(x)
    gidx = _compute_ranks(keys)
    out2d = _build_sc_scatter()(x.reshape(BN, D), gidx.reshape(BN))
    return out2d.reshape(B, N, D)


# speedup vs baseline: 1.3355x; 1.1707x over previous
import functools

import jax
import jax.numpy as jnp
from jax import lax
from jax.experimental import pallas as pl
from jax.experimental.pallas import tpu as pltpu
from jax.experimental.pallas import tpu_sc as plsc

B, N, D = 4, 8192, 1024
BN = B * N

NC, NS = 2, 16
NW = NC * NS
ROWS_PER_W = BN // NW
CHUNK = 32
NCHUNK = ROWS_PER_W // CHUNK


_KA_ROWS = 512


def _keys_body(x_ref, k_ref):
    xb = x_ref[0]
    acc = xb[:, 0:128]
    for j in range(1, 8):
        acc = acc + xb[:, 128 * j:128 * (j + 1)]
    accT = jnp.transpose(acc)
    b = accT[0:8]
    for m in range(1, 16):
        b = b + accT[8 * m:8 * (m + 1)]
    b = b[0:4] + b[4:8]
    b = b[0:2] + b[2:4]
    tot = b[0:1] + b[1:2]
    x0 = jnp.transpose(xb[:, 0:128])[0:1]
    k_ref[0, 0, :] = (x0 + tot)[0]


def _compute_keys(x):
    nblk = N // _KA_ROWS
    out = pl.pallas_call(
        _keys_body,
        grid=(B, nblk),
        in_specs=[pl.BlockSpec((1, _KA_ROWS, D), lambda b, i: (b, i, 0))],
        out_specs=pl.BlockSpec((1, 1, _KA_ROWS), lambda b, i: (b * nblk + i, 0, 0)),
        out_shape=jax.ShapeDtypeStruct((B * nblk, 1, _KA_ROWS), jnp.float32),
    )(x)
    return out.reshape(B, N)


_RB = 1024


def _rank_body(ki_ref, kj_ref, out_ref, acc_ref):
    bidx = pl.program_id(0)
    i = pl.program_id(1)
    j = pl.program_id(2)
    ki = ki_ref[0]
    kj = kj_ref[0]

    @pl.when(j == 0)
    def _():
        acc_ref[...] = jnp.zeros((1, _RB), jnp.float32)

    @pl.when(j < i)
    def _():
        c = (kj <= ki).astype(jnp.float32)
        acc_ref[...] = acc_ref[...] + jnp.sum(c, axis=0, keepdims=True)

    @pl.when(j > i)
    def _():
        c = (kj < ki).astype(jnp.float32)
        acc_ref[...] = acc_ref[...] + jnp.sum(c, axis=0, keepdims=True)

    @pl.when(j == i)
    def _():
        lt = (kj < ki).astype(jnp.float32)
        eq = kj == ki
        qi = lax.broadcasted_iota(jnp.int32, (_RB, _RB), 1)
        qj = lax.broadcasted_iota(jnp.int32, (_RB, _RB), 0)
        tie = jnp.where(eq & (qj < qi), 1.0, 0.0)
        acc_ref[...] = acc_ref[...] + jnp.sum(lt + tie, axis=0, keepdims=True)

    @pl.when(j == pl.num_programs(2) - 1)
    def _():
        out_ref[0] = acc_ref[...].astype(jnp.int32) + bidx * N


def _compute_ranks(keys):
    keys_col = keys.reshape(B, N, 1)
    nb = N // _RB
    keys_row = keys.reshape(B * nb, 1, _RB)
    out = pl.pallas_call(
        _rank_body,
        grid=(B, nb, nb),
        in_specs=[
            pl.BlockSpec((1, 1, _RB), lambda b, i, j: (b * nb + i, 0, 0)),
            pl.BlockSpec((1, _RB, 1), lambda b, i, j: (b, j, 0)),
        ],
        out_specs=pl.BlockSpec((1, 1, _RB), lambda b, i, j: (b * nb + i, 0, 0)),
        out_shape=jax.ShapeDtypeStruct((B * nb, 1, _RB), jnp.int32),
        scratch_shapes=[pltpu.VMEM((1, _RB), jnp.float32)],
    )(keys_row, keys_col)
    return out.reshape(B, N)


@functools.cache
def _build_sc_scatter():
    mesh = plsc.VectorSubcoreMesh(core_axis_name="c", subcore_axis_name="s")

    @functools.partial(
        pl.kernel,
        mesh=mesh,
        out_type=jax.ShapeDtypeStruct((BN, D), jnp.float32),
        scratch_types=[
            pltpu.VMEM((2, CHUNK, D), jnp.float32),
            pltpu.VMEM((NCHUNK, CHUNK), jnp.int32),
            pltpu.SemaphoreType.DMA,
            pltpu.SemaphoreType.DMA,
            pltpu.SemaphoreType.DMA,
            pltpu.SemaphoreType.DMA,
        ],
    )
    def _sc_scatter(x_hbm, gidx_hbm, out_hbm, rows_v, idx_v, si0, si1, so0, so1):
        wid = lax.axis_index("s") * NC + lax.axis_index("c")
        base = wid * ROWS_PER_W
        sin = (si0, si1)
        sout = (so0, so1)

        def in_copy(g, s):
            return pltpu.make_async_copy(
                x_hbm.at[pl.ds(base + g * CHUNK, CHUNK)], rows_v.at[s], sin[s])

        def scat_copy(g, s):
            return pltpu.make_async_copy(
                rows_v.at[s], out_hbm.at[idx_v.at[g]], sout[s])

        pltpu.sync_copy(gidx_hbm.at[wid], idx_v)
        in_copy(0, 0).start()

        def body(t, carry):
            for s in (0, 1):
                g = 2 * t + s
                in_copy(g, s).wait()
                scat_copy(g, s).start()

                @pl.when(g >= 1)
                def _():
                    scat_copy(g - 1, 1 - s).wait()

                @pl.when(g + 1 < NCHUNK)
                def _():
                    in_copy(g + 1, 1 - s).start()

            return carry

        lax.fori_loop(0, NCHUNK // 2, body, 0)
        scat_copy(NCHUNK - 1, (NCHUNK - 1) % 2).wait()

    return _sc_scatter


def kernel(x):
    keys = _compute_keys(x)
    gidx = _compute_ranks(keys)
    out2d = _build_sc_scatter()(x.reshape(BN, D), gidx.reshape(NW, NCHUNK, CHUNK))
    return out2d.reshape(B, N, D)


# speedup vs baseline: 1.6459x; 1.2324x over previous
import functools

import jax
import jax.numpy as jnp
from jax import lax
from jax.experimental import pallas as pl
from jax.experimental.pallas import tpu as pltpu
from jax.experimental.pallas import tpu_sc as plsc

B, N, D = 4, 8192, 1024
BN = B * N

NC, NS = 2, 16
NW = NC * NS
ROWS_PER_W = BN // NW
CHUNK = 32
NCHUNK = ROWS_PER_W // CHUNK


_KA_ROWS = 512


def _keys_body(x_ref, k_ref):
    xb = x_ref[0]
    acc = xb[:, 0:128]
    for j in range(1, 8):
        acc = acc + xb[:, 128 * j:128 * (j + 1)]
    accT = jnp.transpose(acc)
    b = accT[0:8]
    for m in range(1, 16):
        b = b + accT[8 * m:8 * (m + 1)]
    b = b[0:4] + b[4:8]
    b = b[0:2] + b[2:4]
    tot = b[0:1] + b[1:2]
    x0 = jnp.transpose(xb[:, 0:128])[0:1]
    k_ref[0, 0, :] = (x0 + tot)[0]


def _compute_keys(x):
    nblk = N // _KA_ROWS
    out = pl.pallas_call(
        _keys_body,
        grid=(B, nblk),
        in_specs=[pl.BlockSpec((1, _KA_ROWS, D), lambda b, i: (b, i, 0))],
        out_specs=pl.BlockSpec((1, 1, _KA_ROWS), lambda b, i: (b * nblk + i, 0, 0)),
        out_shape=jax.ShapeDtypeStruct((B * nblk, 1, _KA_ROWS), jnp.float32),
    )(x)
    return out.reshape(B, N)


_RB = 1024


_NB = N // _RB
_NT = _NB * (_NB + 1) // 2
_TRI_START = [i * _NB - i * (i - 1) // 2 for i in range(_NB)]


def _tri_ij(t):
    i = jnp.int32(0)
    for k in range(1, _NB):
        i = i + (t >= _TRI_START[k]).astype(jnp.int32)
    start = jnp.int32(0)
    for k in range(1, _NB):
        start = jnp.where(i == k, jnp.int32(_TRI_START[k]), start)
    j = t - start + i
    return i, j


def _rank_body(ki_ref, kj_ref, out_ref, acc_ref):
    bidx = pl.program_id(0)
    t = pl.program_id(1)
    i, j = _tri_ij(t)
    ki = ki_ref[0]
    kj = kj_ref[0]

    @pl.when(t == 0)
    def _():
        acc_ref[...] = jnp.zeros((_NB, _RB), jnp.float32)

    @pl.when(i < j)
    def _():
        c = (kj < ki).astype(jnp.float32)
        s1 = jnp.sum(c, axis=0, keepdims=True)
        acc_ref[pl.ds(i, 1), :] += s1
        cf = c[:, 0:128]
        for a in range(1, 8):
            cf = cf + c[:, 128 * a:128 * (a + 1)]
        ct = jnp.transpose(cf)
        s2 = _RB - jnp.sum(ct, axis=0, keepdims=True)
        acc_ref[pl.ds(j, 1), :] += s2

    @pl.when(i == j)
    def _():
        lt = (kj < ki).astype(jnp.float32)
        eq = kj == ki
        qp = lax.broadcasted_iota(jnp.int32, (_RB, _RB), 1)
        qq = lax.broadcasted_iota(jnp.int32, (_RB, _RB), 0)
        tie = jnp.where(eq & (qq < qp), 1.0, 0.0)
        acc_ref[pl.ds(i, 1), :] += jnp.sum(lt + tie, axis=0, keepdims=True)

    @pl.when(t == _NT - 1)
    def _():
        out_ref[0] = acc_ref[...].astype(jnp.int32) + bidx * N


def _compute_ranks(keys):
    keys_col = keys.reshape(B, N, 1)
    keys_row = keys.reshape(B * _NB, 1, _RB)

    def ki_map(b, t):
        i, _ = _tri_ij(t)
        return b * _NB + i, 0, 0

    def kj_map(b, t):
        _, j = _tri_ij(t)
        return b, j, 0

    out = pl.pallas_call(
        _rank_body,
        grid=(B, _NT),
        in_specs=[
            pl.BlockSpec((1, 1, _RB), ki_map),
            pl.BlockSpec((1, _RB, 1), kj_map),
        ],
        out_specs=pl.BlockSpec((1, _NB, _RB), lambda b, t: (b, 0, 0)),
        out_shape=jax.ShapeDtypeStruct((B, _NB, _RB), jnp.int32),
        scratch_shapes=[pltpu.VMEM((_NB, _RB), jnp.float32)],
    )(keys_row, keys_col)
    return out.reshape(B, N)


@functools.cache
def _build_sc_scatter():
    mesh = plsc.VectorSubcoreMesh(core_axis_name="c", subcore_axis_name="s")

    @functools.partial(
        pl.kernel,
        mesh=mesh,
        out_type=jax.ShapeDtypeStruct((BN, D), jnp.float32),
        scratch_types=[
            pltpu.VMEM((2, CHUNK, D), jnp.float32),
            pltpu.VMEM((NCHUNK, CHUNK), jnp.int32),
            pltpu.SemaphoreType.DMA,
            pltpu.SemaphoreType.DMA,
            pltpu.SemaphoreType.DMA,
            pltpu.SemaphoreType.DMA,
        ],
    )
    def _sc_scatter(x_hbm, gidx_hbm, out_hbm, rows_v, idx_v, si0, si1, so0, so1):
        wid = lax.axis_index("s") * NC + lax.axis_index("c")
        base = wid * ROWS_PER_W
        sin = (si0, si1)
        sout = (so0, so1)

        def in_copy(g, s):
            return pltpu.make_async_copy(
                x_hbm.at[pl.ds(base + g * CHUNK, CHUNK)], rows_v.at[s], sin[s])

        def scat_copy(g, s):
            return pltpu.make_async_copy(
                rows_v.at[s], out_hbm.at[idx_v.at[g]], sout[s])

        pltpu.sync_copy(gidx_hbm.at[wid], idx_v)
        in_copy(0, 0).start()

        def body(t, carry):
            for s in (0, 1):
                g = 2 * t + s
                in_copy(g, s).wait()
                scat_copy(g, s).start()

                @pl.when(g >= 1)
                def _():
                    scat_copy(g - 1, 1 - s).wait()

                @pl.when(g + 1 < NCHUNK)
                def _():
                    in_copy(g + 1, 1 - s).start()

            return carry

        lax.fori_loop(0, NCHUNK // 2, body, 0)
        scat_copy(NCHUNK - 1, (NCHUNK - 1) % 2).wait()

    return _sc_scatter


def kernel(x):
    keys = _compute_keys(x)
    gidx = _compute_ranks(keys)
    out2d = _build_sc_scatter()(x.reshape(BN, D), gidx.reshape(NW, NCHUNK, CHUNK))
    return out2d.reshape(B, N, D)


# speedup vs baseline: 1.8153x; 1.1029x over previous
import functools

import jax
import jax.numpy as jnp
from jax import lax
from jax.experimental import pallas as pl
from jax.experimental.pallas import tpu as pltpu
from jax.experimental.pallas import tpu_sc as plsc

B, N, D = 4, 8192, 1024
BN = B * N

NC, NS = 2, 16
NW = NC * NS
ROWS_PER_W = BN // NW
CHUNK = 32
NCHUNK = ROWS_PER_W // CHUNK


_KA_ROWS = 1024


def _keys_body(x_ref, k_ref):
    xb = x_ref[0]
    acc = xb[:, 0:128]
    for j in range(1, 8):
        acc = acc + xb[:, 128 * j:128 * (j + 1)]
    accT = jnp.transpose(acc)
    b = accT[0:8]
    for m in range(1, 16):
        b = b + accT[8 * m:8 * (m + 1)]
    b = b[0:4] + b[4:8]
    b = b[0:2] + b[2:4]
    tot = b[0:1] + b[1:2]
    x0 = jnp.transpose(xb[:, 0:128])[0:1]
    k_ref[0, 0, :] = (x0 + tot)[0]


def _compute_keys(x):
    nblk = N // _KA_ROWS
    return pl.pallas_call(
        _keys_body,
        grid=(B, nblk),
        in_specs=[pl.BlockSpec((1, _KA_ROWS, D), lambda b, i: (b, i, 0))],
        out_specs=pl.BlockSpec((1, 1, _KA_ROWS), lambda b, i: (b * nblk + i, 0, 0)),
        out_shape=jax.ShapeDtypeStruct((B * nblk, 1, _KA_ROWS), jnp.float32),
    )(x)


_RB = 1024


_NB = N // _RB
_NT = _NB * (_NB + 1) // 2
_TRI_START = [i * _NB - i * (i - 1) // 2 for i in range(_NB)]


def _tri_ij(t):
    i = jnp.int32(0)
    for k in range(1, _NB):
        i = i + (t >= _TRI_START[k]).astype(jnp.int32)
    start = jnp.int32(0)
    for k in range(1, _NB):
        start = jnp.where(i == k, jnp.int32(_TRI_START[k]), start)
    j = t - start + i
    return i, j


def _rank_body(ki_ref, kj_ref, out_ref, acc_ref):
    bidx = pl.program_id(0)
    t = pl.program_id(1)
    i, j = _tri_ij(t)
    ki = ki_ref[0]
    kj = jnp.transpose(kj_ref[0])

    @pl.when(t == 0)
    def _():
        acc_ref[...] = jnp.zeros((_NB, _RB), jnp.float32)

    @pl.when(i < j)
    def _():
        c = (kj < ki).astype(jnp.float32)
        s1 = jnp.sum(c, axis=0, keepdims=True)
        acc_ref[pl.ds(i, 1), :] += s1
        cf = c[:, 0:128]
        for a in range(1, 8):
            cf = cf + c[:, 128 * a:128 * (a + 1)]
        ct = jnp.transpose(cf)
        s2 = _RB - jnp.sum(ct, axis=0, keepdims=True)
        acc_ref[pl.ds(j, 1), :] += s2

    @pl.when(i == j)
    def _():
        lt = (kj < ki).astype(jnp.float32)
        eq = kj == ki
        qp = lax.broadcasted_iota(jnp.int32, (_RB, _RB), 1)
        qq = lax.broadcasted_iota(jnp.int32, (_RB, _RB), 0)
        tie = jnp.where(eq & (qq < qp), 1.0, 0.0)
        acc_ref[pl.ds(i, 1), :] += jnp.sum(lt + tie, axis=0, keepdims=True)

    @pl.when(t == _NT - 1)
    def _():
        out_ref[0] = acc_ref[...].astype(jnp.int32) + bidx * N


def _compute_ranks(keys_row):
    def ki_map(b, t):
        i, _ = _tri_ij(t)
        return b * _NB + i, 0, 0

    def kj_map(b, t):
        _, j = _tri_ij(t)
        return b * _NB + j, 0, 0

    out = pl.pallas_call(
        _rank_body,
        grid=(B, _NT),
        in_specs=[
            pl.BlockSpec((1, 1, _RB), ki_map),
            pl.BlockSpec((1, 1, _RB), kj_map),
        ],
        out_specs=pl.BlockSpec((1, _NB, _RB), lambda b, t: (b, 0, 0)),
        out_shape=jax.ShapeDtypeStruct((B, _NB, _RB), jnp.int32),
        scratch_shapes=[pltpu.VMEM((_NB, _RB), jnp.float32)],
    )(keys_row, keys_row)
    return out.reshape(B, N)


@functools.cache
def _build_sc_scatter():
    mesh = plsc.VectorSubcoreMesh(core_axis_name="c", subcore_axis_name="s")

    @functools.partial(
        pl.kernel,
        mesh=mesh,
        out_type=jax.ShapeDtypeStruct((BN, D), jnp.float32),
        scratch_types=[
            pltpu.VMEM((2, CHUNK, D), jnp.float32),
            pltpu.VMEM((NCHUNK, CHUNK), jnp.int32),
            pltpu.SemaphoreType.DMA,
            pltpu.SemaphoreType.DMA,
            pltpu.SemaphoreType.DMA,
            pltpu.SemaphoreType.DMA,
        ],
    )
    def _sc_scatter(x_hbm, gidx_hbm, out_hbm, rows_v, idx_v, si0, si1, so0, so1):
        wid = lax.axis_index("s") * NC + lax.axis_index("c")
        base = wid * ROWS_PER_W
        sin = (si0, si1)
        sout = (so0, so1)

        def in_copy(g, s):
            return pltpu.make_async_copy(
                x_hbm.at[pl.ds(base + g * CHUNK, CHUNK)], rows_v.at[s], sin[s])

        def scat_copy(g, s):
            return pltpu.make_async_copy(
                rows_v.at[s], out_hbm.at[idx_v.at[g]], sout[s])

        pltpu.sync_copy(gidx_hbm.at[wid], idx_v)
        in_copy(0, 0).start()

        def body(t, carry):
            for s in (0, 1):
                g = 2 * t + s
                in_copy(g, s).wait()
                scat_copy(g, s).start()

                @pl.when(g >= 1)
                def _():
                    scat_copy(g - 1, 1 - s).wait()

                @pl.when(g + 1 < NCHUNK)
                def _():
                    in_copy(g + 1, 1 - s).start()

            return carry

        lax.fori_loop(0, NCHUNK // 2, body, 0)
        scat_copy(NCHUNK - 1, (NCHUNK - 1) % 2).wait()

    return _sc_scatter


def kernel(x):
    keys = _compute_keys(x)
    gidx = _compute_ranks(keys)
    out2d = _build_sc_scatter()(x.reshape(BN, D), gidx.reshape(NW, NCHUNK, CHUNK))
    return out2d.reshape(B, N, D)


# speedup vs baseline: 1.8350x; 1.0109x over previous
import functools

import jax
import jax.numpy as jnp
from jax import lax
from jax.experimental import pallas as pl
from jax.experimental.pallas import tpu as pltpu
from jax.experimental.pallas import tpu_sc as plsc

B, N, D = 4, 8192, 1024
BN = B * N

NC, NS = 2, 16
NW = NC * NS
ROWS_PER_W = BN // NW
CHUNK = 16
NCHUNK = ROWS_PER_W // CHUNK


_KA_ROWS = 1024


def _keys_body(x_ref, k_ref):
    xb = x_ref[0]
    acc = xb[:, 0:128]
    for j in range(1, 8):
        acc = acc + xb[:, 128 * j:128 * (j + 1)]
    accT = jnp.transpose(acc)
    b = accT[0:8]
    for m in range(1, 16):
        b = b + accT[8 * m:8 * (m + 1)]
    b = b[0:4] + b[4:8]
    b = b[0:2] + b[2:4]
    tot = b[0:1] + b[1:2]
    x0 = jnp.transpose(xb[:, 0:128])[0:1]
    k_ref[0, 0, :] = (x0 + tot)[0]


def _compute_keys(x):
    nblk = N // _KA_ROWS
    return pl.pallas_call(
        _keys_body,
        grid=(B, nblk),
        in_specs=[pl.BlockSpec((1, _KA_ROWS, D), lambda b, i: (b, i, 0))],
        out_specs=pl.BlockSpec((1, 1, _KA_ROWS), lambda b, i: (b * nblk + i, 0, 0)),
        out_shape=jax.ShapeDtypeStruct((B * nblk, 1, _KA_ROWS), jnp.float32),
    )(x)


_RB = 1024


_NB = N // _RB
_NT = _NB * (_NB + 1) // 2
_TRI_START = [i * _NB - i * (i - 1) // 2 for i in range(_NB)]


def _tri_ij(t):
    i = jnp.int32(0)
    for k in range(1, _NB):
        i = i + (t >= _TRI_START[k]).astype(jnp.int32)
    start = jnp.int32(0)
    for k in range(1, _NB):
        start = jnp.where(i == k, jnp.int32(_TRI_START[k]), start)
    j = t - start + i
    return i, j


def _rank_body(ki_ref, kj_ref, out_ref, acc_ref):
    bidx = pl.program_id(0)
    t = pl.program_id(1)
    i, j = _tri_ij(t)
    ki = ki_ref[0]
    kj = jnp.transpose(kj_ref[0])

    @pl.when(t == 0)
    def _():
        acc_ref[...] = jnp.zeros((_NB, _RB), jnp.float32)

    @pl.when(i < j)
    def _():
        c = (kj < ki).astype(jnp.float32)
        s1 = jnp.sum(c, axis=0, keepdims=True)
        acc_ref[pl.ds(i, 1), :] += s1
        cf = c[:, 0:128]
        for a in range(1, 8):
            cf = cf + c[:, 128 * a:128 * (a + 1)]
        ct = jnp.transpose(cf)
        s2 = _RB - jnp.sum(ct, axis=0, keepdims=True)
        acc_ref[pl.ds(j, 1), :] += s2

    @pl.when(i == j)
    def _():
        lt = (kj < ki).astype(jnp.float32)
        eq = kj == ki
        qp = lax.broadcasted_iota(jnp.int32, (_RB, _RB), 1)
        qq = lax.broadcasted_iota(jnp.int32, (_RB, _RB), 0)
        tie = jnp.where(eq & (qq < qp), 1.0, 0.0)
        acc_ref[pl.ds(i, 1), :] += jnp.sum(lt + tie, axis=0, keepdims=True)

    @pl.when(t == _NT - 1)
    def _():
        out_ref[0] = acc_ref[...].astype(jnp.int32) + bidx * N


def _compute_ranks(keys_row):
    def ki_map(b, t):
        i, _ = _tri_ij(t)
        return b * _NB + i, 0, 0

    def kj_map(b, t):
        _, j = _tri_ij(t)
        return b * _NB + j, 0, 0

    out = pl.pallas_call(
        _rank_body,
        grid=(B, _NT),
        in_specs=[
            pl.BlockSpec((1, 1, _RB), ki_map),
            pl.BlockSpec((1, 1, _RB), kj_map),
        ],
        out_specs=pl.BlockSpec((1, _NB, _RB), lambda b, t: (b, 0, 0)),
        out_shape=jax.ShapeDtypeStruct((B, _NB, _RB), jnp.int32),
        scratch_shapes=[pltpu.VMEM((_NB, _RB), jnp.float32)],
    )(keys_row, keys_row)
    return out.reshape(B, N)


@functools.cache
def _build_sc_scatter():
    mesh = plsc.VectorSubcoreMesh(core_axis_name="c", subcore_axis_name="s")

    nslot = 4

    @functools.partial(
        pl.kernel,
        mesh=mesh,
        out_type=jax.ShapeDtypeStruct((BN, D), jnp.float32),
        scratch_types=[
            pltpu.VMEM((nslot, CHUNK, D), jnp.float32),
            pltpu.VMEM((NCHUNK, CHUNK), jnp.int32),
            pltpu.SemaphoreType.DMA,
            pltpu.SemaphoreType.DMA,
            pltpu.SemaphoreType.DMA,
            pltpu.SemaphoreType.DMA,
            pltpu.SemaphoreType.DMA,
            pltpu.SemaphoreType.DMA,
            pltpu.SemaphoreType.DMA,
            pltpu.SemaphoreType.DMA,
        ],
    )
    def _sc_scatter(x_hbm, gidx_hbm, out_hbm, rows_v, idx_v,
                    si0, si1, si2, si3, so0, so1, so2, so3):
        wid = lax.axis_index("s") * NC + lax.axis_index("c")
        base = wid * ROWS_PER_W
        sin = (si0, si1, si2, si3)
        sout = (so0, so1, so2, so3)

        def in_copy(g, s):
            return pltpu.make_async_copy(
                x_hbm.at[pl.ds(base + g * CHUNK, CHUNK)], rows_v.at[s], sin[s])

        def scat_copy(g, s):
            return pltpu.make_async_copy(
                rows_v.at[s], out_hbm.at[idx_v.at[g]], sout[s])

        pltpu.sync_copy(gidx_hbm.at[wid], idx_v)
        in_copy(0, 0).start()
        in_copy(1, 1).start()

        def body(t, carry):
            for s in range(nslot):
                g = nslot * t + s
                in_copy(g, s).wait()
                scat_copy(g, s).start()

                @pl.when(g >= 2)
                def _():
                    scat_copy(g - 2, (s + 2) % nslot).wait()

                @pl.when(g + 2 < NCHUNK)
                def _():
                    in_copy(g + 2, (s + 2) % nslot).start()

            return carry

        lax.fori_loop(0, NCHUNK // nslot, body, 0)
        scat_copy(NCHUNK - 2, (NCHUNK - 2) % nslot).wait()
        scat_copy(NCHUNK - 1, (NCHUNK - 1) % nslot).wait()

    return _sc_scatter


def kernel(x):
    keys = _compute_keys(x)
    gidx = _compute_ranks(keys)
    out2d = _build_sc_scatter()(x.reshape(BN, D), gidx.reshape(NW, NCHUNK, CHUNK))
    return out2d.reshape(B, N, D)
